# baseline (speedup 1.0000x reference)
import functools

import jax
import jax.numpy as jnp
from jax import lax
from jax.experimental import pallas as pl
from jax.experimental.pallas import tpu as pltpu
from jax.experimental.pallas import tpu_sc as plsc


RATIO = 0.5

_NP = 10240
_NW = 32
_NPW = 320
_D = 256
_BI = 256


def _rank_kernel(score_col_ref, score_row_ref, rank_ref):
    i0 = pl.program_id(0) * _BI
    si = score_col_ref[...]
    sj = score_row_ref[0:1, :]
    gt = sj > si
    eq = sj == si
    jj = lax.broadcasted_iota(jnp.int32, (_BI, _NP), 1)
    ii = i0 + lax.broadcasted_iota(jnp.int32, (_BI, _NP), 0)
    wins = gt | (eq & (jj < ii))
    wb = wins.astype(jnp.bfloat16)
    ones = jnp.ones((_NP, 1), jnp.bfloat16)
    cnt = jnp.dot(wb, ones, preferred_element_type=jnp.float32)
    rank_ref[...] = cnt.astype(jnp.int32)


def _scatter_body(rank_hbm, xs_hbm, out_hbm, idx_v, xr_v, sem):
    wid = lax.axis_index("s") * 2 + lax.axis_index("c")
    pltpu.sync_copy(rank_hbm.at[wid], idx_v)
    pltpu.sync_copy(xs_hbm.at[pl.ds(wid * _NPW, _NPW), :], xr_v)
    pltpu.async_copy(xr_v, out_hbm.at[idx_v], sem).wait()


_scatter_out = functools.partial(
    pl.kernel,
    mesh=plsc.VectorSubcoreMesh(core_axis_name="c", subcore_axis_name="s"),
    out_type=jax.ShapeDtypeStruct((_NP, _D), jnp.float32),
    scratch_types=[
        pltpu.VMEM((_NPW,), jnp.int32),
        pltpu.VMEM((_NPW, _D), jnp.float32),
        pltpu.SemaphoreType.DMA,
    ],
)(_scatter_body)


def _topk_gather(score, x, n_nodes, k):
    score_pad = jnp.full((_NP,), -2.0, jnp.float32).at[:n_nodes].set(score)
    score_col = score_pad.reshape(_NP, 1)
    score_row = jnp.broadcast_to(score_pad.reshape(1, _NP), (8, _NP))

    rank = pl.pallas_call(
        _rank_kernel,
        grid=(_NP // _BI,),
        in_specs=[
            pl.BlockSpec((_BI, 1), lambda i: (i, 0)),
            pl.BlockSpec((8, _NP), lambda i: (0, 0)),
        ],
        out_specs=pl.BlockSpec((_BI, 1), lambda i: (i, 0)),
        out_shape=jax.ShapeDtypeStruct((_NP, 1), jnp.int32),
    )(score_col, score_row)

    xs = jnp.zeros((_NP, _D), jnp.float32).at[:n_nodes].set(x * score[:, None])
    out = _scatter_out(rank.reshape(_NW, _NPW), xs)
    return out[:k]


def kernel(x, edge_index, batch, W_l, b_l, W_r):
    n = x.shape[0]
    row, col = edge_index[0], edge_index[1]

    agg = jnp.zeros_like(x).at[col].add(x[row])
    score = (agg @ W_l.T + b_l + x @ W_r.T).reshape(-1)
    score = jnp.tanh(score)

    k = (n + 1) // 2
    x_out = _topk_gather(score, x, n, k)
    batch_out = jnp.zeros((k,), jnp.int32)
    return (x_out, batch_out)


# speedup vs baseline: 1.0122x; 1.0122x over previous
import functools

import jax
import jax.numpy as jnp
from jax import lax
from jax.experimental import pallas as pl
from jax.experimental.pallas import tpu as pltpu
from jax.experimental.pallas import tpu_sc as plsc


RATIO = 0.5

_NP = 10240
_NW = 32
_NPW = 320
_D = 256
_BI = 256


def _rank_kernel(score_col_ref, score_row_ref, rank_ref):
    i0 = pl.program_id(0) * _BI
    si = score_col_ref[...]
    sj = score_row_ref[0:1, :]
    gt = sj > si
    eq = sj == si
    jj = lax.broadcasted_iota(jnp.int32, (_BI, _NP), 1)
    ii = i0 + lax.broadcasted_iota(jnp.int32, (_BI, _NP), 0)
    wins = gt | (eq & (jj < ii))
    cnt = jnp.sum(jnp.where(wins, 1.0, 0.0), axis=1, keepdims=True)
    rank_ref[...] = cnt.astype(jnp.int32)


_BS = 512


def _score_kernel(agg_ref, x_ref, wl_ref, wr_ref, b_ref, score_ref, xs_ref):
    i0 = pl.program_id(0) * _BS
    a = agg_ref[...]
    xx = x_ref[...]
    wl = wl_ref[...]
    wr = wr_ref[...]
    bb = b_ref[0:1, 0:1]
    d1 = jnp.dot(a, wl, preferred_element_type=jnp.float32)
    d2 = jnp.dot(xx, wr, preferred_element_type=jnp.float32)
    s = jnp.tanh((d1 + bb) + d2)
    rid = i0 + lax.broadcasted_iota(jnp.int32, (_BS, 1), 0)
    s = jnp.where(rid < 10000, s, -2.0)
    score_ref[...] = s
    xs_ref[...] = xx * s


def _score_xs(agg_pad, x_pad, wl_t, wr_t, b8):
    return pl.pallas_call(
        _score_kernel,
        grid=(_NP // _BS,),
        in_specs=[
            pl.BlockSpec((_BS, _D), lambda i: (i, 0)),
            pl.BlockSpec((_BS, _D), lambda i: (i, 0)),
            pl.BlockSpec((_D, 1), lambda i: (0, 0)),
            pl.BlockSpec((_D, 1), lambda i: (0, 0)),
            pl.BlockSpec((8, 128), lambda i: (0, 0)),
        ],
        out_specs=[
            pl.BlockSpec((_BS, 1), lambda i: (i, 0)),
            pl.BlockSpec((_BS, _D), lambda i: (i, 0)),
        ],
        out_shape=[
            jax.ShapeDtypeStruct((_NP, 1), jnp.float32),
            jax.ShapeDtypeStruct((_NP, _D), jnp.float32),
        ],
    )(agg_pad, x_pad, wl_t, wr_t, b8)


def _scatter_body(rank_hbm, xs_hbm, out_hbm, idx_v, xr_v, sem):
    wid = lax.axis_index("s") * 2 + lax.axis_index("c")
    pltpu.sync_copy(rank_hbm.at[wid], idx_v)
    pltpu.sync_copy(xs_hbm.at[pl.ds(wid * _NPW, _NPW), :], xr_v)
    pltpu.async_copy(xr_v, out_hbm.at[idx_v], sem).wait()


_scatter_out = functools.partial(
    pl.kernel,
    mesh=plsc.VectorSubcoreMesh(core_axis_name="c", subcore_axis_name="s"),
    out_type=jax.ShapeDtypeStruct((_NP, _D), jnp.float32),
    scratch_types=[
        pltpu.VMEM((_NPW,), jnp.int32),
        pltpu.VMEM((_NPW, _D), jnp.float32),
        pltpu.SemaphoreType.DMA,
    ],
)(_scatter_body)


def _topk_gather(score_col, xs, k):
    score_row = jnp.broadcast_to(score_col.reshape(1, _NP), (8, _NP))

    rank = pl.pallas_call(
        _rank_kernel,
        grid=(_NP // _BI,),
        in_specs=[
            pl.BlockSpec((_BI, 1), lambda i: (i, 0)),
            pl.BlockSpec((8, _NP), lambda i: (0, 0)),
        ],
        out_specs=pl.BlockSpec((_BI, 1), lambda i: (i, 0)),
        out_shape=jax.ShapeDtypeStruct((_NP, 1), jnp.int32),
    )(score_col, score_row)

    out = _scatter_out(rank.reshape(_NW, _NPW), xs)
    return out[:k]


def kernel(x, edge_index, batch, W_l, b_l, W_r):
    n = x.shape[0]
    row, col = edge_index[0], edge_index[1]

    agg = jnp.zeros_like(x).at[col].add(x[row])

    pad = jnp.zeros((_NP - n, _D), jnp.float32)
    agg_pad = jnp.concatenate([agg, pad], axis=0)
    x_pad = jnp.concatenate([x, pad], axis=0)
    b8 = jnp.broadcast_to(b_l.reshape(1, 1), (8, 128))
    score_col, xs = _score_xs(agg_pad, x_pad, W_l.reshape(_D, 1),
                              W_r.reshape(_D, 1), b8)

    k = (n + 1) // 2
    x_out = _topk_gather(score_col, xs, k)
    batch_out = jnp.zeros((k,), jnp.int32)
    return (x_out, batch_out)


# speedup vs baseline: 1.0174x; 1.0052x over previous
import functools

import jax
import jax.numpy as jnp
from jax import lax
from jax.experimental import pallas as pl
from jax.experimental.pallas import tpu as pltpu
from jax.experimental.pallas import tpu_sc as plsc


RATIO = 0.5

_NP = 10240
_NW = 32
_NPW = 320
_D = 256
_BI = 256


def _rank_kernel(score_col_ref, score_row_ref, rank_ref):
    i0 = pl.program_id(0) * _BI
    si = score_col_ref[...]
    sj = score_row_ref[0:1, :]
    gt = sj > si
    eq = sj == si
    jj = lax.broadcasted_iota(jnp.int32, (_BI, _NP), 1)
    ii = i0 + lax.broadcasted_iota(jnp.int32, (_BI, _NP), 0)
    wins = gt | (eq & (jj < ii))
    cnt = jnp.sum(jnp.where(wins, 1.0, 0.0), axis=1, keepdims=True)
    rank_ref[...] = cnt.astype(jnp.int32)


_BS = 512


def _score_kernel(agg_ref, x_ref, wl_ref, wr_ref, b_ref, score_ref, xs_ref):
    i0 = pl.program_id(0) * _BS
    a = agg_ref[...]
    xx = x_ref[...]
    wl = wl_ref[...]
    wr = wr_ref[...]
    bb = b_ref[0:1, 0:1]
    d1 = jnp.dot(a, wl, preferred_element_type=jnp.float32)
    d2 = jnp.dot(xx, wr, preferred_element_type=jnp.float32)
    s = jnp.tanh((d1 + bb) + d2)
    rid = i0 + lax.broadcasted_iota(jnp.int32, (_BS, 1), 0)
    s = jnp.where(rid < 10000, s, -2.0)
    score_ref[...] = s
    xs_ref[...] = xx * s


def _score_xs(agg_pad, x_pad, wl_t, wr_t, b8):
    return pl.pallas_call(
        _score_kernel,
        grid=(_NP // _BS,),
        in_specs=[
            pl.BlockSpec((_BS, _D), lambda i: (i, 0)),
            pl.BlockSpec((_BS, _D), lambda i: (i, 0)),
            pl.BlockSpec((_D, 1), lambda i: (0, 0)),
            pl.BlockSpec((_D, 1), lambda i: (0, 0)),
            pl.BlockSpec((8, 128), lambda i: (0, 0)),
        ],
        out_specs=[
            pl.BlockSpec((_BS, 1), lambda i: (i, 0)),
            pl.BlockSpec((_BS, _D), lambda i: (i, 0)),
        ],
        out_shape=[
            jax.ShapeDtypeStruct((_NP, 1), jnp.float32),
            jax.ShapeDtypeStruct((_NP, _D), jnp.float32),
        ],
    )(agg_pad, x_pad, wl_t, wr_t, b8)


def _scatter_body(rank_hbm, xs_hbm, out_hbm, idx_v, xr_v, sem):
    wid = lax.axis_index("s") * 2 + lax.axis_index("c")
    pltpu.sync_copy(rank_hbm.at[wid], idx_v)
    pltpu.sync_copy(xs_hbm.at[pl.ds(wid * _NPW, _NPW), :], xr_v)
    pltpu.async_copy(xr_v, out_hbm.at[idx_v], sem).wait()


_scatter_out = functools.partial(
    pl.kernel,
    mesh=plsc.VectorSubcoreMesh(core_axis_name="c", subcore_axis_name="s"),
    out_type=jax.ShapeDtypeStruct((_NP, _D), jnp.float32),
    scratch_types=[
        pltpu.VMEM((_NPW,), jnp.int32),
        pltpu.VMEM((_NPW, _D), jnp.float32),
        pltpu.SemaphoreType.DMA,
    ],
)(_scatter_body)


def _topk_gather(score_col, xs, k):
    score_row = jnp.broadcast_to(score_col.reshape(1, _NP), (8, _NP))

    rank = pl.pallas_call(
        _rank_kernel,
        grid=(_NP // _BI,),
        in_specs=[
            pl.BlockSpec((_BI, 1), lambda i: (i, 0)),
            pl.BlockSpec((8, _NP), lambda i: (0, 0)),
        ],
        out_specs=pl.BlockSpec((_BI, 1), lambda i: (i, 0)),
        out_shape=jax.ShapeDtypeStruct((_NP, 1), jnp.int32),
    )(score_col, score_row)

    out = _scatter_out(rank.reshape(_NW, _NPW), xs)
    return out[:k]


def kernel(x, edge_index, batch, W_l, b_l, W_r):
    n = x.shape[0]
    row, col = edge_index[0], edge_index[1]

    agg = jnp.zeros_like(x).at[col].add(x[row])

    b8 = jnp.broadcast_to(b_l.reshape(1, 1), (8, 128))
    score_col, xs = _score_xs(agg, x, W_l.reshape(_D, 1),
                              W_r.reshape(_D, 1), b8)

    k = (n + 1) // 2
    x_out = _topk_gather(score_col, xs, k)
    batch_out = jnp.zeros((k,), jnp.int32)
    return (x_out, batch_out)
